# Initial kernel scaffold; baseline (speedup 1.0000x reference)
import jax
import jax.numpy as jnp
from jax.experimental import pallas as pl


def kernel(preds, gts, normals):
    raise NotImplementedError("write your pallas kernel here")


# speedup vs baseline: 12.4513x; 12.4513x over previous
import math

import jax
import jax.numpy as jnp
from jax.experimental import pallas as pl

_KNN = 8
_INV_SP2 = 1.0 / (0.03 ** 2)
_INV_C2 = 1.0 / (1.0 - math.cos(math.radians(15.0)))


def _loss_kernel(preds_ref, gts_ref, normals_ref, gg_ref, c_ref, out_ref):
    q = preds_ref[0]
    g = gts_ref[0]
    nrm = normals_ref[0]
    gg = gg_ref[0]
    c = c_ref[0]

    nq = q.shape[0]
    m = g.shape[0]
    f32 = jnp.float32
    hi = jax.lax.Precision.HIGHEST

    qq = jnp.sum(q * q, axis=1, keepdims=True)

    dims_t = (((1,), (1,)), ((), ()))
    qg = jax.lax.dot_general(q.astype(jnp.bfloat16), g.astype(jnp.bfloat16),
                             dims_t, preferred_element_type=f32)
    d = (qq + gg) - 2.0 * qg

    big = jnp.float32(jnp.inf)
    iota = jax.lax.broadcasted_iota(jnp.int32, (nq, m), 1)

    m0 = jnp.min(d, axis=1, keepdims=True)
    eq = d == m0
    idx0 = jnp.min(jnp.where(eq, iota, m), axis=1, keepdims=True)
    onehot0 = (iota == idx0).astype(f32)
    avail = ~eq
    for _ in range(_KNN - 1):
        masked = jnp.where(avail, d, big)
        mv = jnp.min(masked, axis=1, keepdims=True)
        avail = avail & (masked != mv)
    sel = ~avail

    e = jax.lax.dot_general(onehot0, nrm, (((1,), (0,)), ((), ())),
                            preferred_element_type=f32, precision=hi)

    inner_n = jax.lax.dot_general(e, nrm, dims_t,
                                  preferred_element_type=f32, precision=hi)

    p = jax.lax.dot_general(q, nrm, dims_t,
                            preferred_element_type=f32, precision=hi)
    ip = jnp.abs(p - c)

    w = jnp.exp(d * (-_INV_SP2)) * jnp.exp((inner_n - 1.0) * _INV_C2)
    w = jnp.where(sel, w, 0.0)

    num = jnp.sum(w * ip, axis=1, keepdims=True)
    den = jnp.sum(w, axis=1, keepdims=True)
    tile_sum = jnp.sum(num / den).reshape(1, 1)

    @pl.when(jnp.logical_and(pl.program_id(0) == 0, pl.program_id(1) == 0))
    def _init():
        out_ref[:, :] = jnp.zeros((1, 1), jnp.float32)

    out_ref[:, :] += tile_sum


def kernel(preds, gts, normals):
    b, n, _ = preds.shape
    m = gts.shape[1]
    nq = 256

    gg = jnp.sum(gts * gts, axis=-1)[:, None, :]
    c = jnp.sum(gts * normals, axis=-1)[:, None, :]

    out = pl.pallas_call(
        _loss_kernel,
        grid=(b, n // nq),
        in_specs=[
            pl.BlockSpec((1, nq, 3), lambda bi, i: (bi, i, 0)),
            pl.BlockSpec((1, m, 3), lambda bi, i: (bi, 0, 0)),
            pl.BlockSpec((1, m, 3), lambda bi, i: (bi, 0, 0)),
            pl.BlockSpec((1, 1, m), lambda bi, i: (bi, 0, 0)),
            pl.BlockSpec((1, 1, m), lambda bi, i: (bi, 0, 0)),
        ],
        out_specs=pl.BlockSpec((1, 1), lambda bi, i: (0, 0)),
        out_shape=jax.ShapeDtypeStruct((1, 1), jnp.float32),
    )(preds, gts, normals, gg, c)
    return out[0, 0]


# speedup vs baseline: 24.5643x; 1.9728x over previous
import math

import jax
import jax.numpy as jnp
from jax.experimental import pallas as pl

_KNN = 8
_INV_SP2 = 1.0 / (0.03 ** 2)
_INV_C2 = 1.0 / (1.0 - math.cos(math.radians(15.0)))


def _loss_kernel(preds_ref, gts_ref, normals_ref, nrmt_ref, gg_ref, c_ref,
                 out_ref):
    q = preds_ref[0]
    g = gts_ref[0]
    nrm = normals_ref[0]
    nrmt = nrmt_ref[0]
    gg = gg_ref[0]
    c = c_ref[0]

    f32 = jnp.float32
    hi = jax.lax.Precision.HIGHEST

    qq = jnp.sum(q * q, axis=1, keepdims=True)

    dims_t = (((1,), (1,)), ((), ()))
    qg = jax.lax.dot_general(q.astype(jnp.bfloat16), g.astype(jnp.bfloat16),
                             dims_t, preferred_element_type=f32)
    d = (qq + gg) - 2.0 * qg

    big = jnp.float32(jnp.inf)

    t = jnp.min(d, axis=1, keepdims=True)
    eq = d == t
    e = jnp.concatenate(
        [jnp.min(jnp.where(eq, nrmt[k:k + 1, :], big), axis=1, keepdims=True)
         for k in range(3)], axis=1)

    for _ in range(_KNN - 1):
        t = jnp.min(jnp.where(d > t, d, big), axis=1, keepdims=True)

    inner_n = jax.lax.dot_general(e, nrm, dims_t,
                                  preferred_element_type=f32, precision=hi)

    p = jax.lax.dot_general(q, nrm, dims_t,
                            preferred_element_type=f32, precision=hi)
    ip = jnp.abs(p - c)

    w = jnp.where(d <= t, jnp.exp(d * (-_INV_SP2) + (inner_n - 1.0) * _INV_C2),
                  0.0)

    num = jnp.sum(w * ip, axis=1, keepdims=True)
    den = jnp.sum(w, axis=1, keepdims=True)
    tile_sum = jnp.sum(num / den).reshape(1, 1)

    @pl.when(jnp.logical_and(pl.program_id(0) == 0, pl.program_id(1) == 0))
    def _init():
        out_ref[:, :] = jnp.zeros((1, 1), jnp.float32)

    out_ref[:, :] += tile_sum


def kernel(preds, gts, normals):
    b, n, _ = preds.shape
    m = gts.shape[1]
    nq = 256

    nrmt = jnp.swapaxes(normals, 1, 2)
    gg = jnp.sum(gts * gts, axis=-1)[:, None, :]
    c = jnp.sum(gts * normals, axis=-1)[:, None, :]

    out = pl.pallas_call(
        _loss_kernel,
        grid=(b, n // nq),
        in_specs=[
            pl.BlockSpec((1, nq, 3), lambda bi, i: (bi, i, 0)),
            pl.BlockSpec((1, m, 3), lambda bi, i: (bi, 0, 0)),
            pl.BlockSpec((1, m, 3), lambda bi, i: (bi, 0, 0)),
            pl.BlockSpec((1, 3, m), lambda bi, i: (bi, 0, 0)),
            pl.BlockSpec((1, 1, m), lambda bi, i: (bi, 0, 0)),
            pl.BlockSpec((1, 1, m), lambda bi, i: (bi, 0, 0)),
        ],
        out_specs=pl.BlockSpec((1, 1), lambda bi, i: (0, 0)),
        out_shape=jax.ShapeDtypeStruct((1, 1), jnp.float32),
    )(preds, gts, normals, nrmt, gg, c)
    return out[0, 0]


# speedup vs baseline: 39.0069x; 1.5880x over previous
import math

import jax
import jax.numpy as jnp
from jax.experimental import pallas as pl

_KNN = 8
_INV_SP2 = 1.0 / (0.03 ** 2)
_INV_C2 = 1.0 / (1.0 - math.cos(math.radians(15.0)))


def _loss_kernel(preds_ref, rhs_ref, normals_ref, nrmt_ref, gg_ref, c_ref,
                 out_ref):
    q = preds_ref[0]
    rhs = rhs_ref[0]
    nrm = normals_ref[0]
    nrmt = nrmt_ref[0]
    gg = gg_ref[0]
    c = c_ref[0]

    m = nrm.shape[0]
    f32 = jnp.float32
    bf16 = jnp.bfloat16

    qq = jnp.sum(q * q, axis=1, keepdims=True)

    q_hi = q.astype(bf16)
    q_lo = (q - q_hi.astype(f32)).astype(bf16)
    lhs = jnp.concatenate([q_hi, q_lo, q_hi], axis=1)

    dims_t = (((1,), (1,)), ((), ()))
    both = jax.lax.dot_general(lhs, rhs, dims_t,
                               preferred_element_type=f32)
    qg = both[:, :m]
    p = both[:, m:]

    d = (qq + gg) - 2.0 * qg
    ip = jnp.abs(p - c)

    big = jnp.float32(jnp.inf)

    t = jnp.min(d, axis=1, keepdims=True)
    eq = d == t
    e = jnp.concatenate(
        [jnp.min(jnp.where(eq, nrmt[k:k + 1, :], big), axis=1, keepdims=True)
         for k in range(3)], axis=1)

    for _ in range(_KNN - 1):
        t = jnp.min(jnp.where(d > t, d, big), axis=1, keepdims=True)

    inner_n = jax.lax.dot_general(e.astype(bf16), nrm.astype(bf16), dims_t,
                                  preferred_element_type=f32)

    w = jnp.where(d <= t, jnp.exp(d * (-_INV_SP2) + (inner_n - 1.0) * _INV_C2),
                  0.0)

    num = jnp.sum(w * ip, axis=1, keepdims=True)
    den = jnp.sum(w, axis=1, keepdims=True)
    tile_sum = jnp.sum(num / den).reshape(1, 1)

    @pl.when(jnp.logical_and(pl.program_id(0) == 0, pl.program_id(1) == 0))
    def _init():
        out_ref[:, :] = jnp.zeros((1, 1), jnp.float32)

    out_ref[:, :] += tile_sum


def kernel(preds, gts, normals):
    b, n, _ = preds.shape
    m = gts.shape[1]
    nq = 256
    f32 = jnp.float32
    bf16 = jnp.bfloat16

    n_hi = normals.astype(bf16)
    n_lo = (normals - n_hi.astype(f32)).astype(bf16)
    zeros6 = jnp.zeros((b, m, 6), dtype=bf16)
    rhs = jnp.concatenate([
        jnp.concatenate([gts.astype(bf16), zeros6], axis=2),
        jnp.concatenate([n_hi, n_hi, n_lo], axis=2),
    ], axis=1)

    nrmt = jnp.swapaxes(normals, 1, 2)
    gg = jnp.sum(gts * gts, axis=-1)[:, None, :]
    c = jnp.sum(gts * normals, axis=-1)[:, None, :]

    out = pl.pallas_call(
        _loss_kernel,
        grid=(b, n // nq),
        in_specs=[
            pl.BlockSpec((1, nq, 3), lambda bi, i: (bi, i, 0)),
            pl.BlockSpec((1, 2 * m, 9), lambda bi, i: (bi, 0, 0)),
            pl.BlockSpec((1, m, 3), lambda bi, i: (bi, 0, 0)),
            pl.BlockSpec((1, 3, m), lambda bi, i: (bi, 0, 0)),
            pl.BlockSpec((1, 1, m), lambda bi, i: (bi, 0, 0)),
            pl.BlockSpec((1, 1, m), lambda bi, i: (bi, 0, 0)),
        ],
        out_specs=pl.BlockSpec((1, 1), lambda bi, i: (0, 0)),
        out_shape=jax.ShapeDtypeStruct((1, 1), jnp.float32),
    )(preds, rhs, normals, nrmt, gg, c)
    return out[0, 0]


# speedup vs baseline: 43.5667x; 1.1169x over previous
import math

import jax
import jax.numpy as jnp
from jax.experimental import pallas as pl

_KNN = 8
_INV_SP2 = 1.0 / (0.03 ** 2)
_INV_C2 = 1.0 / (1.0 - math.cos(math.radians(15.0)))


def _loss_kernel(preds_ref, rhs_ref, nrmb_ref, out_ref):
    q = preds_ref[0]
    rhs = rhs_ref[0]
    nrmb = nrmb_ref[0]

    m = nrmb.shape[0]
    nq = q.shape[0]
    f32 = jnp.float32
    bf16 = jnp.bfloat16

    qq = jnp.sum(q * q, axis=1, keepdims=True)

    q_hi = q.astype(bf16)
    q_lo = (q - q_hi.astype(f32)).astype(bf16)
    ones3 = jnp.ones((nq, 3), dtype=bf16)
    lhs = jnp.concatenate([q_hi, q_lo, q_hi, ones3], axis=1)

    dims_t = (((1,), (1,)), ((), ()))
    both = jax.lax.dot_general(lhs, rhs, dims_t,
                               preferred_element_type=f32)
    dr = both[:, :m]
    ip = jnp.abs(both[:, m:])

    big = jnp.float32(jnp.inf)

    t = jnp.min(dr, axis=1, keepdims=True)
    eq = (dr == t).astype(bf16)
    e = jax.lax.dot_general(eq, nrmb, (((1,), (0,)), ((), ())),
                            preferred_element_type=f32)

    for _ in range(_KNN - 1):
        t = jnp.min(jnp.where(dr > t, dr, big), axis=1, keepdims=True)

    inner_n = jax.lax.dot_general(e.astype(bf16), nrmb, dims_t,
                                  preferred_element_type=f32)

    cc = qq * (-_INV_SP2) - _INV_C2
    w = jnp.where(dr <= t,
                  jnp.exp(dr * (-_INV_SP2) + inner_n * _INV_C2 + cc), 0.0)

    num = jnp.sum(w * ip, axis=1, keepdims=True)
    den = jnp.sum(w, axis=1, keepdims=True)
    tile_sum = jnp.sum(num / den).reshape(1, 1)

    @pl.when(jnp.logical_and(pl.program_id(0) == 0, pl.program_id(1) == 0))
    def _init():
        out_ref[:, :] = jnp.zeros((1, 1), jnp.float32)

    out_ref[:, :] += tile_sum


def _split3(x, bf16, f32):
    h = x.astype(bf16)
    r1 = x - h.astype(f32)
    mid = r1.astype(bf16)
    lo = (r1 - mid.astype(f32)).astype(bf16)
    return h, mid, lo


def kernel(preds, gts, normals):
    b, n, _ = preds.shape
    m = gts.shape[1]
    nq = 256
    f32 = jnp.float32
    bf16 = jnp.bfloat16

    n_hi = normals.astype(bf16)
    n_lo = (normals - n_hi.astype(f32)).astype(bf16)

    gg = jnp.sum(gts * gts, axis=-1)[..., None]
    gg_h, gg_m, gg_l = _split3(gg, bf16, f32)
    c = jnp.sum(gts * normals, axis=-1)[..., None]
    c_h, c_m, _ = _split3(c, bf16, f32)

    zeros6 = jnp.zeros((b, m, 6), dtype=bf16)
    zeros1 = jnp.zeros((b, m, 1), dtype=bf16)
    rhs = jnp.concatenate([
        jnp.concatenate([(-2.0 * gts).astype(bf16), zeros6,
                         gg_h, gg_m, gg_l], axis=2),
        jnp.concatenate([n_hi, n_hi, n_lo, -c_h, -c_m, zeros1],
                        axis=2),
    ], axis=1)

    out = pl.pallas_call(
        _loss_kernel,
        grid=(b, n // nq),
        in_specs=[
            pl.BlockSpec((1, nq, 3), lambda bi, i: (bi, i, 0)),
            pl.BlockSpec((1, 2 * m, 12), lambda bi, i: (bi, 0, 0)),
            pl.BlockSpec((1, m, 3), lambda bi, i: (bi, 0, 0)),
        ],
        out_specs=pl.BlockSpec((1, 1), lambda bi, i: (0, 0)),
        out_shape=jax.ShapeDtypeStruct((1, 1), jnp.float32),
    )(preds, rhs, normals.astype(bf16))
    return out[0, 0]
